# Initial kernel scaffold; baseline (speedup 1.0000x reference)
import jax
import jax.numpy as jnp
from jax.experimental import pallas as pl


def kernel(user, item, keyword_ids, W_user, W_kw, W1, b1, W2, b2, W3, b3):
    raise NotImplementedError("write your pallas kernel here")


# speedup vs baseline: 5.9034x; 5.9034x over previous
import functools

import jax
import jax.numpy as jnp
from jax import lax
from jax.experimental import pallas as pl
from jax.experimental.pallas import tpu as pltpu
from jax.experimental.pallas import tpu_sc as plsc

NUM_USERS = 1000000
NUM_KEYWORDS = 100000
D = 64
B = 16384
HIST = 50

NC = 2
NS = 16
NW = NC * NS
BPW = B // NW
NCH = BPW // 128


def _sc_body(u3_r, kwt_r, wu_r, wk_r, uout_r, ksum_r,
             uidx, kidx, urows, acc, sem_u, sem_k):
    c_id = lax.axis_index("c")
    s_id = lax.axis_index("s")
    wid = s_id * NC + c_id
    base = wid * BPW
    cb = wid * NCH

    pltpu.sync_copy(u3_r.at[pl.ds(cb, NCH)], uidx)
    pltpu.sync_copy(kwt_r.at[:, pl.ds(cb, NCH), :], kidx)

    for c in range(NCH):
        pltpu.async_copy(wu_r.at[uidx.at[c]],
                         urows.at[pl.ds(c * 128, 128)], sem_u)
        pltpu.async_copy(wk_r.at[kidx.at[0, c]],
                         acc.at[pl.ds(c * 128, 128)], sem_k)
    pltpu.make_async_copy(wu_r.at[pl.ds(0, BPW)], urows, sem_u).wait()
    pltpu.make_async_copy(wk_r.at[pl.ds(0, BPW)], acc, sem_k).wait()

    pltpu.sync_copy(urows, uout_r.at[pl.ds(base, BPW)])

    def fire(j, carry):
        for c in range(NCH):
            pltpu.async_copy(wk_r.at[kidx.at[j, c]],
                             acc.at[pl.ds(c * 128, 128)], sem_k, add=True)
        return carry

    lax.fori_loop(1, HIST, fire, 0)

    def drain(j, carry):
        pltpu.make_async_copy(wk_r.at[pl.ds(0, BPW)], acc, sem_k).wait()
        return carry

    lax.fori_loop(1, HIST, drain, 0)

    pltpu.sync_copy(acc, ksum_r.at[pl.ds(base, BPW)])


def _sc_gather(user3, kwt3, W_user, W_kw):
    mesh = plsc.VectorSubcoreMesh(core_axis_name="c", subcore_axis_name="s")
    return pl.kernel(
        _sc_body,
        out_type=[
            jax.ShapeDtypeStruct((B, D), jnp.float32),
            jax.ShapeDtypeStruct((B, D), jnp.float32),
        ],
        mesh=mesh,
        scratch_types=[
            pltpu.VMEM((NCH, 128), jnp.int32),
            pltpu.VMEM((HIST, NCH, 128), jnp.int32),
            pltpu.VMEM((BPW, D), jnp.float32),
            pltpu.VMEM((BPW, D), jnp.float32),
            pltpu.SemaphoreType.DMA,
            pltpu.SemaphoreType.DMA,
        ],
        compiler_params=pltpu.CompilerParams(use_tc_tiling_on_sc=False),
    )(user3, kwt3, W_user, W_kw)


BLK = 2048


def _tc_body(u_r, ks_r, kw_r, w0_r, w1_r, b1_r, w2_r, b2_r, w3_r, b3_r, out_r):
    u = u_r[...]
    ks = ks_r[...]
    kw = kw_r[...]
    n0 = jnp.sum((kw == 0).astype(jnp.float32), axis=1, keepdims=True)
    cnt = float(HIST) - n0
    row0 = w0_r[...]
    kvec = jnp.where(cnt > 0.0,
                     (ks - n0 * row0) / jnp.maximum(cnt, 1.0),
                     0.0)
    x = jnp.concatenate([u, kvec], axis=1)
    h = lax.dot_general(x, w1_r[...], (((1,), (1,)), ((), ())),
                        preferred_element_type=jnp.float32)
    h = jnp.maximum(h + b1_r[...], 0.0)
    h = lax.dot_general(h, w2_r[...], (((1,), (1,)), ((), ())),
                        preferred_element_type=jnp.float32)
    h = jnp.maximum(h + b2_r[...], 0.0)
    out_r[...] = jnp.sum(h * w3_r[...], axis=1) + b3_r[0, 0]


def _tc_mlp(u_vec, ksum, keyword_ids, w0, W1, b1, W2, b2, W3, b3):
    grid = B // BLK
    full = lambda shape: pl.BlockSpec(shape, lambda i: (0,) * len(shape))
    return pl.pallas_call(
        _tc_body,
        grid=(grid,),
        in_specs=[
            pl.BlockSpec((BLK, D), lambda i: (i, 0)),
            pl.BlockSpec((BLK, D), lambda i: (i, 0)),
            pl.BlockSpec((BLK, HIST), lambda i: (i, 0)),
            full((1, D)),
            full((128, 2 * D)),
            full((1, 128)),
            full((D, 128)),
            full((1, D)),
            full((1, D)),
            full((1, 1)),
        ],
        out_specs=pl.BlockSpec((BLK,), lambda i: (i,)),
        out_shape=jax.ShapeDtypeStruct((B,), jnp.float32),
    )(u_vec, ksum, keyword_ids, w0, W1, b1, W2, b2, W3, b3)


@jax.jit
def kernel(user, item, keyword_ids, W_user, W_kw, W1, b1, W2, b2, W3, b3):
    del item
    user3 = user.astype(jnp.int32).reshape(NW * NCH, 128)
    kwt3 = keyword_ids.astype(jnp.int32).T.reshape(HIST, NW * NCH, 128)
    u_vec, ksum = _sc_gather(user3, kwt3, W_user, W_kw)
    return _tc_mlp(u_vec, ksum, keyword_ids.astype(jnp.int32),
                   W_kw[0:1], W1, b1.reshape(1, 128), W2, b2.reshape(1, D),
                   W3, b3.reshape(1, 1))
